# baseline (speedup 1.0000x reference)
import functools

import jax
import jax.numpy as jnp
from jax import lax
from jax.experimental import pallas as pl
from jax.experimental.pallas import tpu as pltpu
from jax.experimental.pallas import tpu_sc as plsc

f32 = jnp.float32
_HI = jax.lax.Precision.HIGHEST


def _dot(a, b):
    return jnp.dot(a, b, preferred_element_type=f32, precision=_HI)

N = 50000
E = 800000
H = 64
HH = H // 2
PK = 16

N_PAD = 50176
NP = N_PAD // PK
EP = E // PK
NV = N // PK

BPN = 392
BPE = 1000

NSUB = 16
K = 256
EPT = 50176
CH_A = 196
CH_B = 185
NBUF = 3
ROWS_PT = N_PAD // NSUB
ZROWS = 98


def _embed_kernel(x_ref, w_ref, olo_ref, ohi_ref):
    x = x_ref[...]
    kdim = w_ref.shape[0]
    groups = x.shape[1] // kdim
    los, his = [], []
    for j in range(groups):
        z = jnp.dot(x[:, j * kdim:(j + 1) * kdim], w_ref[...],
                    preferred_element_type=f32)
        los.append(z[:, :HH])
        his.append(z[:, HH:])
    zlo = jnp.concatenate(los, axis=1)
    zhi = jnp.concatenate(his, axis=1)
    olo_ref[...] = zlo.reshape(zlo.shape[0] * zlo.shape[1])
    ohi_ref[...] = zhi.reshape(zhi.shape[0] * zhi.shape[1])


def _embed(x, w, blk):
    rows = x.shape[0]
    outw = (x.shape[1] // w.shape[0]) * HH
    grid = rows // blk
    return pl.pallas_call(
        _embed_kernel,
        grid=(grid,),
        in_specs=[
            pl.BlockSpec((blk, x.shape[1]), lambda i: (i, 0)),
            pl.BlockSpec(w.shape, lambda i: (0, 0)),
        ],
        out_specs=[
            pl.BlockSpec((blk * outw,), lambda i: (i,)),
            pl.BlockSpec((blk * outw,), lambda i: (i,)),
        ],
        out_shape=[
            jax.ShapeDtypeStruct((rows * outw,), f32),
            jax.ShapeDtypeStruct((rows * outw,), f32),
        ],
    )(x, w)


def _mlp_core(hl, hh, al, ah, w1, w2, b1, b2):
    zl = hl + al
    zh = hh + ah
    outs_l, outs_h = [], []
    for j in range(PK):
        sl = slice(j * HH, (j + 1) * HH)
        hj = jnp.concatenate([hl[:, sl], hh[:, sl]], axis=1)
        zj = jnp.concatenate([zl[:, sl], zh[:, sl]], axis=1)
        y = jnp.maximum(jnp.dot(zj, w1, preferred_element_type=f32)
                        + b1, 0.0)
        y = jnp.dot(y, w2, preferred_element_type=f32) + b2
        hn = jnp.maximum(y, 0.0) + hj
        outs_l.append(hn[:, :HH])
        outs_h.append(hn[:, HH:])
    return (jnp.concatenate(outs_l, axis=1),
            jnp.concatenate(outs_h, axis=1))


def _mlp_kernel(hl_ref, hh_ref, al_ref, ah_ref,
                w1_ref, w2_ref, bb_ref, ol_ref, oh_ref):
    hl = hl_ref[...].reshape(BPN, 512)
    hh = hh_ref[...].reshape(BPN, 512)
    al = al_ref[...].reshape(BPN, 512)
    ah = ah_ref[...].reshape(BPN, 512)
    hnl, hnh = _mlp_core(hl, hh, al, ah, w1_ref[...], w2_ref[...],
                         bb_ref[0:1, :], bb_ref[1:2, :])
    ol_ref[...] = hnl.reshape(BPN * 512)
    oh_ref[...] = hnh.reshape(BPN * 512)


def _mlp_specs():
    flat = pl.BlockSpec((BPN * 512,), lambda i: (i,))
    w = pl.BlockSpec((H, H), lambda i: (0, 0))
    return ([flat, flat, flat, flat, w, w,
             pl.BlockSpec((2, H), lambda i: (0, 0))], flat)


def _mlp(hl, hh, al, ah, kws, bb):
    in_specs, flat = _mlp_specs()
    return pl.pallas_call(
        _mlp_kernel,
        grid=(NP // BPN,),
        in_specs=in_specs,
        out_specs=[flat, flat],
        out_shape=[
            jax.ShapeDtypeStruct((N_PAD * HH,), f32),
            jax.ShapeDtypeStruct((N_PAD * HH,), f32),
        ],
    )(hl, hh, al, ah, *kws, bb)


def _mlp_readout_kernel(hl_ref, hh_ref, al_ref, ah_ref,
                        w1_ref, w2_ref,
                        bb_ref, fold_ref, wo_ref, bo_ref,
                        out_ref, accl_ref, acch_ref):
    i = pl.program_id(0)
    hl = hl_ref[...].reshape(BPN, 512)
    hh = hh_ref[...].reshape(BPN, 512)
    al = al_ref[...].reshape(BPN, 512)
    ah = ah_ref[...].reshape(BPN, 512)
    hnl, hnh = _mlp_core(hl, hh, al, ah, w1_ref[...], w2_ref[...],
                         bb_ref[0:1, :], bb_ref[1:2, :])
    row = lax.broadcasted_iota(jnp.int32, (BPN, 1), 0) + i * BPN
    valid = row < NV
    pl_ = jnp.sum(jnp.where(valid, hnl, 0.0), axis=0, keepdims=True)
    ph_ = jnp.sum(jnp.where(valid, hnh, 0.0), axis=0, keepdims=True)

    @pl.when(i == 0)
    def _():
        accl_ref[...] = pl_
        acch_ref[...] = ph_

    @pl.when(i > 0)
    def _():
        accl_ref[...] = accl_ref[...] + pl_
        acch_ref[...] = acch_ref[...] + ph_

    @pl.when(i == pl.num_programs(0) - 1)
    def _():
        accl = accl_ref[...]
        acch = acch_ref[...]
        tl = accl[:, 0:HH]
        th = acch[:, 0:HH]
        for j in range(1, PK):
            tl = tl + accl[:, j * HH:(j + 1) * HH]
            th = th + acch[:, j * HH:(j + 1) * HH]
        out_ref[...] = (jnp.sum(tl * wo_ref[0:1, :], axis=1, keepdims=True)
                        + jnp.sum(th * wo_ref[1:2, :], axis=1, keepdims=True)
                        + bo_ref[...])


def _mlp_readout(hl, hh, al, ah, kws, bb, fold, wo2, bo_r):
    in_specs, _ = _mlp_specs()
    in_specs = in_specs + [
        pl.BlockSpec((512, HH), lambda i: (0, 0)),
        pl.BlockSpec((2, HH), lambda i: (0, 0)),
        pl.BlockSpec((1, 1), lambda i: (0, 0)),
    ]
    return pl.pallas_call(
        _mlp_readout_kernel,
        grid=(NP // BPN,),
        in_specs=in_specs,
        out_specs=pl.BlockSpec((1, 1), lambda i: (0, 0)),
        out_shape=jax.ShapeDtypeStruct((1, 1), f32),
        scratch_shapes=[pltpu.VMEM((1, 512), f32), pltpu.VMEM((1, 512), f32)],
    )(hl, hh, al, ah, *kws, bb, fold, wo2, bo_r)


def _edge_half(s, h_ref, e_ref, ei_ref, out_ref,
               sib, ebuf, zbuf, spacc, sem_l, sem_g, sem_s):
    def zrow(i, _):
        zbuf[i, pl.ds(0, 16)] = jnp.zeros((16,), f32)
        zbuf[i, pl.ds(16, 16)] = jnp.zeros((16,), f32)
        return _
    lax.fori_loop(0, ZROWS, zrow, None, unroll=4)
    base = s * ROWS_PT
    for j in range(ROWS_PT // ZROWS):
        pltpu.sync_copy(zbuf, spacc.at[pl.ds(base + j * ZROWS, ZROWS)])
    plsc.subcore_barrier()

    n = jnp.where(s < NSUB - 1, CH_A, CH_B)
    ebase = s * EPT

    def load_copies(i, b):
        lo = ebase + i * K
        return (pltpu.make_async_copy(e_ref.at[pl.ds(lo, K)], ebuf.at[b],
                                      sem_l),
                pltpu.make_async_copy(ei_ref.at[pl.ds(lo, 128)],
                                      sib.at[b, 0, 0], sem_l),
                pltpu.make_async_copy(ei_ref.at[pl.ds(lo + 128, 128)],
                                      sib.at[b, 0, 1], sem_l),
                pltpu.make_async_copy(ei_ref.at[pl.ds(E + lo, 128)],
                                      sib.at[b, 1, 0], sem_l),
                pltpu.make_async_copy(ei_ref.at[pl.ds(E + lo + 128, 128)],
                                      sib.at[b, 1, 1], sem_l))

    def start_loads(i, b):
        for d in load_copies(i, b):
            d.start()

    def wait_loads(i, b):
        for d in load_copies(i, b):
            d.wait()

    def gather_descs(i, b):
        return (pltpu.make_async_copy(h_ref.at[sib.at[b, 0, 0]],
                                      ebuf.at[b, pl.ds(0, 128)], sem_g),
                pltpu.make_async_copy(h_ref.at[sib.at[b, 0, 1]],
                                      ebuf.at[b, pl.ds(128, 128)], sem_g))

    def scatter_descs(b):
        return (pltpu.make_async_copy(ebuf.at[b, pl.ds(0, 128)],
                                      spacc.at[sib.at[b, 1, 0]], sem_s),
                pltpu.make_async_copy(ebuf.at[b, pl.ds(128, 128)],
                                      spacc.at[sib.at[b, 1, 1]], sem_s))

    def start_gathers(i, b):
        for d in gather_descs(i, b):
            d.start(add=True)

    def wait_gathers(i, b):
        for d in gather_descs(i, b):
            d.wait()

    def start_scatters(b):
        for d in scatter_descs(b):
            d.start(add=True)

    def wait_scatters(b):
        for d in scatter_descs(b):
            d.wait()

    start_loads(0, 0)
    wait_loads(0, 0)
    start_gathers(0, 0)
    start_loads(1, 1)

    def group(g, carry):
        for b in range(NBUF):
            i = g * NBUF + b

            @pl.when(i < n)
            def _():
                wait_gathers(i, b)

                @pl.when(i + 1 < n)
                def _():
                    bn = (b + 1) % NBUF
                    wait_loads(i + 1, bn)
                    start_gathers(i + 1, bn)

                def rrow(r, _):
                    ebuf[b, r, pl.ds(0, 16)] = jnp.maximum(
                        ebuf[b, r, pl.ds(0, 16)], 0.0)
                    ebuf[b, r, pl.ds(16, 16)] = jnp.maximum(
                        ebuf[b, r, pl.ds(16, 16)], 0.0)
                    return _
                lax.fori_loop(0, K, rrow, None, unroll=8)

                @pl.when(i > 0)
                def _():
                    wait_scatters((b - 1) % NBUF)

                @pl.when(i + 2 < n)
                def _():
                    start_loads(i + 2, (b + 2) % NBUF)

                start_scatters(b)
        return carry
    lax.fori_loop(0, (CH_A + NBUF - 1) // NBUF, group, None)
    @pl.when(s < NSUB - 1)
    def _():
        wait_scatters(0)

    @pl.when(s == NSUB - 1)
    def _():
        wait_scatters(1)
    plsc.subcore_barrier()
    for j in range(ROWS_PT // ZROWS):
        sl = pl.ds(base + j * ZROWS, ZROWS)
        pltpu.sync_copy(spacc.at[sl], out_ref.at[sl])


def _edge_body(hlo, hhi, elo, ehi, ei, agg_lo, agg_hi,
               sib, ebuf, zbuf, spacc, sem_l, sem_g, sem_s):
    c = lax.axis_index("c")
    s = lax.axis_index("s")

    @pl.when(c == 0)
    def _():
        _edge_half(s, hlo, elo, ei, agg_lo,
                   sib, ebuf, zbuf, spacc, sem_l, sem_g, sem_s)

    @pl.when(c == 1)
    def _():
        _edge_half(s, hhi, ehi, ei, agg_hi,
                   sib, ebuf, zbuf, spacc, sem_l, sem_g, sem_s)


def _edge_call(hlo, hhi, elo, ehi, ei_flat):
    mesh = plsc.VectorSubcoreMesh(core_axis_name="c", subcore_axis_name="s")
    fn = pl.kernel(
        _edge_body,
        out_type=(
            jax.ShapeDtypeStruct((N_PAD, HH), f32),
            jax.ShapeDtypeStruct((N_PAD, HH), f32),
        ),
        mesh=mesh,
        scratch_types=[
            pltpu.VMEM((NBUF, 2, 2, 128), jnp.int32),
            pltpu.VMEM((NBUF, K, HH), f32),
            pltpu.VMEM((ZROWS, HH), f32),
            pltpu.VMEM_SHARED((N_PAD, HH), f32),
            pltpu.SemaphoreType.DMA,
            pltpu.SemaphoreType.DMA,
            pltpu.SemaphoreType.DMA,
        ],
        compiler_params=pltpu.CompilerParams(use_tc_tiling_on_sc=False),
    )
    return fn(hlo, hhi, elo, ehi, ei_flat)


def kernel(feat, eweight, edge_index, Wn, We, params, Wo, bo):
    Wn_p = jnp.pad(Wn, ((0, H - Wn.shape[0]), (0, 0)))
    fold = jnp.kron(jnp.ones((PK, 1), f32), jnp.eye(HH, dtype=f32))
    wo2 = Wo.reshape(2, HH)
    bo_r = bo.reshape(1, 1)

    featp = jnp.pad(feat, ((0, N_PAD - N), (0, H - feat.shape[1])))
    featp = featp.reshape(NP, PK * H)
    ewp = eweight.reshape(E * 6 // 192, 192)
    ei_flat = edge_index.reshape(2 * E)

    h_lo, h_hi = _embed(featp, Wn_p, BPN)
    e_lo, e_hi = _embed(ewp, We, BPE)
    e_lo2 = e_lo.reshape(E, HH)
    e_hi2 = e_hi.reshape(E, HH)

    out = None
    for li, (W1, b1, W2, b2) in enumerate(params):
        kws = (W1, W2)
        bb = jnp.stack([b1, b2])
        agg_lo, agg_hi = _edge_call(h_lo.reshape(N_PAD, HH),
                                    h_hi.reshape(N_PAD, HH),
                                    e_lo2, e_hi2, ei_flat)
        al = agg_lo.reshape(N_PAD * HH)
        ah = agg_hi.reshape(N_PAD * HH)
        if li < 3:
            h_lo, h_hi = _mlp(h_lo, h_hi, al, ah, kws, bb)
        else:
            out = _mlp_readout(h_lo, h_hi, al, ah, kws, bb, fold, wo2, bo_r)
    return out
